# baseline (speedup 1.0000x reference)
import jax
import jax.numpy as jnp
from jax.experimental import pallas as pl
from jax.experimental.pallas import tpu as pltpu

N = 10000
CONCAT = 4
IN_CH = 128
HID = 16
OUT_CH = CONCAT * HID
NUM_CLASSES = 40
BM = 200
NI = N // (2 * BM)
BMY = 2000
NY = N // BMY


def _lahgcn_kernel(x_ref, hga_ref, hgb_ref, w1_ref, b1_ref, w2_ref, b2_ref,
                   out_ref, y_ref, z_ref):
    t = pl.program_id(0)

    @pl.when(t < NY)
    def _compute_y():
        for k in range(CONCAT):
            yk = jnp.dot(x_ref[k], w1_ref[k],
                         preferred_element_type=jnp.float32)
            yk = yk + b1_ref[k:k + 1, :]
            y_ref[pl.ds(t * BMY, BMY), pl.ds(k * HID, HID)] = yk

    @pl.when((t >= NY) & (t < NY + NI))
    def _compute_z():
        i = t - NY
        for half, hg_ref in ((0, hga_ref), (1, hgb_ref)):
            h = jnp.dot(hg_ref[...], y_ref[...],
                        preferred_element_type=jnp.float32)
            h = jnp.maximum(h, 0.0)
            z = jnp.dot(h, w2_ref[...],
                        preferred_element_type=jnp.float32) + b2_ref[0:1, :]
            z_ref[pl.ds((2 * i + half) * BM, BM), :] = z

    @pl.when(t >= NY + NI)
    def _compute_out():
        out_ref[pl.ds(0, BM), :] = jnp.dot(
            hga_ref[...], z_ref[...], preferred_element_type=jnp.float32)
        out_ref[pl.ds(BM, BM), :] = jnp.dot(
            hgb_ref[...], z_ref[...], preferred_element_type=jnp.float32)


def kernel(x_list, hg, W1, b1, W2, b2):
    b2_2d = b2.reshape(1, NUM_CLASSES)
    T = NY + 2 * NI

    def hg_row(t, half):
        i = jnp.where(t < NY, 0,
                      jnp.where(t < NY + NI, t - NY, t - NY - NI))
        return 2 * i + half

    return pl.pallas_call(
        _lahgcn_kernel,
        grid=(T,),
        in_specs=[
            pl.BlockSpec((CONCAT, BMY, IN_CH),
                         lambda t: (0, jnp.minimum(t, NY - 1), 0)),
            pl.BlockSpec((BM, N), lambda t: (hg_row(t, 0), 0)),
            pl.BlockSpec((BM, N), lambda t: (hg_row(t, 1), 0)),
            pl.BlockSpec((CONCAT, IN_CH, HID), lambda t: (0, 0, 0)),
            pl.BlockSpec((CONCAT, HID), lambda t: (0, 0)),
            pl.BlockSpec((OUT_CH, NUM_CLASSES), lambda t: (0, 0)),
            pl.BlockSpec((1, NUM_CLASSES), lambda t: (0, 0)),
        ],
        out_specs=pl.BlockSpec(
            (2 * BM, NUM_CLASSES),
            lambda t: (jnp.where(t >= NY + NI, t - NY - NI, 0), 0)),
        out_shape=jax.ShapeDtypeStruct((N, NUM_CLASSES), jnp.float32),
        scratch_shapes=[
            pltpu.VMEM((N, OUT_CH), jnp.float32),
            pltpu.VMEM((N, NUM_CLASSES), jnp.float32),
        ],
        compiler_params=pltpu.CompilerParams(
            dimension_semantics=("arbitrary",),
        ),
    )(x_list, hg, hg, W1, b1, W2, b2_2d)


# speedup vs baseline: 1.0431x; 1.0431x over previous
import jax
import jax.numpy as jnp
from jax.experimental import pallas as pl
from jax.experimental.pallas import tpu as pltpu

N = 10000
CONCAT = 4
IN_CH = 128
HID = 16
OUT_CH = CONCAT * HID
NUM_CLASSES = 40
BM = 400
NI = N // BM
BMY = 2000
NY = N // BMY


def _lahgcn_kernel(x_ref, hg_ref, w1_ref, b1_ref, w2_ref, b2_ref,
                   out_ref, y_ref, z_ref):
    t = pl.program_id(0)

    @pl.when(t < NY)
    def _compute_y():
        for k in range(CONCAT):
            yk = jnp.dot(x_ref[k], w1_ref[k],
                         preferred_element_type=jnp.float32)
            yk = yk + b1_ref[k:k + 1, :]
            y_ref[pl.ds(t * BMY, BMY), pl.ds(k * HID, HID)] = (
                yk.astype(jnp.bfloat16))

    @pl.when((t >= NY) & (t < NY + NI))
    def _compute_z():
        i = t - NY
        h = jnp.dot(hg_ref[...].astype(jnp.bfloat16), y_ref[...],
                    preferred_element_type=jnp.float32)
        h = jnp.maximum(h, 0.0)
        z = jnp.dot(h, w2_ref[...],
                    preferred_element_type=jnp.float32) + b2_ref[0:1, :]
        z_ref[pl.ds(i * BM, BM), :] = z.astype(jnp.bfloat16)

    @pl.when(t >= NY + NI)
    def _compute_out():
        out_ref[...] = jnp.dot(hg_ref[...].astype(jnp.bfloat16), z_ref[...],
                               preferred_element_type=jnp.float32)


def kernel(x_list, hg, W1, b1, W2, b2):
    b2_2d = b2.reshape(1, NUM_CLASSES)
    T = NY + 2 * NI

    def hg_idx(t):
        return (jnp.where(t < NY, 0,
                          jnp.where(t < NY + NI, t - NY, t - NY - NI)), 0)

    return pl.pallas_call(
        _lahgcn_kernel,
        grid=(T,),
        in_specs=[
            pl.BlockSpec((CONCAT, BMY, IN_CH),
                         lambda t: (0, jnp.minimum(t, NY - 1), 0)),
            pl.BlockSpec((BM, N), hg_idx),
            pl.BlockSpec((CONCAT, IN_CH, HID), lambda t: (0, 0, 0)),
            pl.BlockSpec((CONCAT, HID), lambda t: (0, 0)),
            pl.BlockSpec((OUT_CH, NUM_CLASSES), lambda t: (0, 0)),
            pl.BlockSpec((1, NUM_CLASSES), lambda t: (0, 0)),
        ],
        out_specs=pl.BlockSpec(
            (BM, NUM_CLASSES),
            lambda t: (jnp.where(t >= NY + NI, t - NY - NI, 0), 0)),
        out_shape=jax.ShapeDtypeStruct((N, NUM_CLASSES), jnp.float32),
        scratch_shapes=[
            pltpu.VMEM((N, OUT_CH), jnp.bfloat16),
            pltpu.VMEM((N, NUM_CLASSES), jnp.bfloat16),
        ],
        compiler_params=pltpu.CompilerParams(
            dimension_semantics=("arbitrary",),
        ),
    )(x_list, hg, W1, b1, W2, b2_2d)
